# baseline (speedup 1.0000x reference)
import jax
import jax.numpy as jnp
from jax import lax
from jax.experimental import pallas as pl
from jax.experimental.pallas import tpu as pltpu, tpu_sc as plsc

N = 65536
B = 8
M = N // B
FEATURE_DIM = 256
MAX_TOKENS = 128
TOKEN_DIM = 768

ROWS = 1024

_bf = jnp.bfloat16
_f32 = jnp.float32
_i32 = jnp.int32
_u32 = jnp.uint32


def _bdot(x, w):
    return jnp.dot(x.astype(_bf), w, preferred_element_type=_f32)


def _mlp_body(cf_ref, feat_ref, lng_ref, lnb_ref,
              ws1_ref, bs1_ref, ws2_ref, bs2_ref,
              w1_ref, b1_ref,
              w2_ref, b2_ref, w3_ref, b3_ref, w4_ref, b4_ref,
              wi1_ref, bi1_ref, wi2_ref, bi2_ref,
              pf_ref, imp_ref, key_ref):
    x4 = cf_ref[...]
    mu = jnp.mean(x4, axis=1, keepdims=True)
    var = jnp.mean((x4 - mu) ** 2, axis=1, keepdims=True)
    cf = (x4 - mu) / jnp.sqrt(var + 1e-5) * lng_ref[...] + lnb_ref[...]

    sp = jnp.maximum(_bdot(cf, ws1_ref[...]) + bs1_ref[...], 0.0)
    sp = _bdot(sp, ws2_ref[...]) + bs2_ref[...]

    cat = jnp.concatenate([feat_ref[...].astype(_bf), sp.astype(_bf)], axis=1)
    h = jnp.maximum(jnp.dot(cat, w1_ref[...], preferred_element_type=_f32)
                    + b1_ref[...], 0.0)
    h = jnp.maximum(_bdot(h, w2_ref[...]) + b2_ref[...], 0.0)
    h = jnp.maximum(_bdot(h, w3_ref[...]) + b3_ref[...], 0.0)
    pf = _bdot(h, w4_ref[...]) + b4_ref[...]
    pf_ref[...] = pf

    t = jnp.maximum(_bdot(pf, wi1_ref[...]) + bi1_ref[...], 0.0)
    imp = _bdot(t, wi2_ref[...]) + bi2_ref[...]
    imp_ref[...] = imp

    bits = jax.lax.bitcast_convert_type(imp, jnp.int32)
    key = bits ^ (jnp.int32(-0x80000000) | (bits >> 31))
    key_ref[...] = key


def _run_mlp(cf4, features, ln_g, ln_b, Ws1, bs1, Ws2, bs2,
             Wm1, bm1, Wm2, bm2, Wm3, bm3, Wm4, bm4, Wi1, bi1, Wi2, bi2):
    grid = N // ROWS
    row_spec = lambda width: pl.BlockSpec((ROWS, width), lambda i: (i, 0))
    full = lambda a: pl.BlockSpec(a.shape, lambda i: (0,) * a.ndim)

    weights = [ln_g[None, :], ln_b[None, :],
               Ws1.T.astype(_bf), bs1[None, :],
               Ws2.T.astype(_bf), bs2[None, :],
               Wm1.T.astype(_bf), bm1[None, :],
               Wm2.T.astype(_bf), bm2[None, :],
               Wm3.T.astype(_bf), bm3[None, :],
               Wm4.T.astype(_bf), bm4[None, :],
               Wi1.T.astype(_bf), bi1[None, :],
               Wi2.T.astype(_bf), bi2[None, :]]

    pf, imp, key = pl.pallas_call(
        _mlp_body,
        grid=(grid,),
        in_specs=[row_spec(4), row_spec(FEATURE_DIM)] + [full(w) for w in weights],
        out_specs=[row_spec(TOKEN_DIM), row_spec(1), row_spec(1)],
        out_shape=[
            jax.ShapeDtypeStruct((N, TOKEN_DIM), jnp.float32),
            jax.ShapeDtypeStruct((N, 1), jnp.float32),
            jax.ShapeDtypeStruct((N, 1), jnp.int32),
        ],
    )(cf4, features, *weights)
    return pf, imp[:, 0], key[:, 0]


_VECS = M // 16


def _sc_body(key_hbm, pf_hbm, c128_hbm,
             toks_hbm, cents128_hbm,
             keys_v, selidx_v, eq_v, seltime_v, selkey_v,
             sortidx_v, crows_v, rows_v, sum_v,
             sem):
    wid = lax.axis_index("s") * 2 + lax.axis_index("c")

    @pl.when(wid < B)
    def _():
        lanes = lax.iota(_i32, 16)
        zeros16 = jnp.zeros((16,), _i32)
        ones16 = jnp.ones((16,), _i32)
        lane0 = lanes == 0

        def _sload(ref, p):
            return ref[pl.ds(p, 16)][0]

        def _sstore(ref, p, val, dtype):
            v = ref[pl.ds(p, 16)]
            ref[pl.ds(p, 16)] = jnp.where(lane0, jnp.full((16,), val, dtype),
                                          v)

        def _cstore(ref, p, val, flag, dtype):
            v = ref[pl.ds(p, 16)]
            new0 = jnp.where(flag > 0, val, v[0])
            ref[pl.ds(p, 16)] = jnp.where(lane0, jnp.full((16,), new0, dtype),
                                          v)

        def _vsum(vec):
            sum_v[pl.ds(0, 16)] = vec
            s = jnp.int32(0)
            for q in range(16):
                s = s + sum_v[pl.ds(q, 16)][0]
            return s

        def _popcnt(m):
            return _vsum(jnp.where(m, ones16, zeros16))

        pltpu.sync_copy(key_hbm.at[pl.ds(wid * M, M)],
                        keys_v.at[pl.ds(0, M)])

        def _count_ge(t):
            def body(i, cnt):
                kv = plsc.bitcast(keys_v[pl.ds(i * 16, 16)], _u32)
                return cnt + jnp.where(kv >= t, ones16, zeros16)
            return _vsum(lax.fori_loop(0, _VECS, body, zeros16))

        T = jnp.uint32(0)
        for bit in range(31, -1, -1):
            Ttry = T | jnp.uint32(1 << bit)
            T = jnp.where(_count_ge(Ttry) >= MAX_TOKENS, Ttry, T)
        need = MAX_TOKENS - _count_ge(T + jnp.uint32(1))

        def _compact(i, carry):
            kv = plsc.bitcast(keys_v[pl.ds(i * 16, 16)], _u32)
            m_gt = kv > T
            m_eq = kv == T
            any_hit = _popcnt(m_gt | m_eq) > 0

            def _hit(c):
                n_gt, n_eq = c
                sum_v[pl.ds(32, 16)] = jnp.where(m_gt, ones16, zeros16)
                sum_v[pl.ds(48, 16)] = jnp.where(m_eq, ones16, zeros16)

                def _lane(q, cc):
                    ng, ne = cc
                    fg = sum_v[pl.ds(32 + q, 16)][0]
                    fe = sum_v[pl.ds(48 + q, 16)][0]
                    iq = wid * M + i * 16 + q
                    _cstore(selidx_v, ng, iq, fg, _i32)
                    _cstore(eq_v, jnp.minimum(ne, 132), iq, fe, _i32)
                    return (ng + fg, ne + fe)
                return lax.fori_loop(0, 16, _lane, c)
            return lax.cond(any_hit, _hit, lambda c: c, carry)
        n_gt, _ = lax.fori_loop(0, _VECS, _compact, (jnp.int32(0),
                                                     jnp.int32(0)))

        def _fill_eq(j, _):
            _sstore(selidx_v, n_gt + j, _sload(eq_v, j), _i32)
            return 0
        lax.fori_loop(0, need, _fill_eq, 0)

        pltpu.async_copy(c128_hbm.at[selidx_v.at[pl.ds(0, MAX_TOKENS)]],
                         crows_v, sem).wait()

        def _lookup(p, _):
            tp = crows_v[p, pl.ds(0, 16)][3]
            _sstore(seltime_v, p, tp, _f32)
            _sstore(selkey_v, p,
                    _sload(keys_v, _sload(selidx_v, p) - wid * M), _i32)
            return 0
        lax.fori_loop(0, MAX_TOKENS, _lookup, 0)

        st = [seltime_v[pl.ds(16 * a, 16)] for a in range(8)]
        sk = [plsc.bitcast(selkey_v[pl.ds(16 * a, 16)], _u32)
              for a in range(8)]
        si = [selidx_v[pl.ds(16 * a, 16)] for a in range(8)]

        def _place(p, acc):
            tb = jnp.full((16,), _sload(seltime_v, p), _f32)
            kb = plsc.bitcast(jnp.full((16,), _sload(selkey_v, p), _i32),
                              _u32)
            ib = jnp.full((16,), _sload(selidx_v, p), _i32)
            rank = jnp.int32(0)
            for a in range(8):
                before = (st[a] < tb) | (
                    (st[a] == tb) & ((sk[a] > kb) | ((sk[a] == kb)
                                                     & (si[a] < ib))))
                rank = rank + _popcnt(before)
            rb = jnp.full((16,), rank, _i32)
            return tuple(
                jnp.where(rb == (16 * o + lanes), ib, acc[o])
                for o in range(8))
        acc = lax.fori_loop(0, MAX_TOKENS, _place,
                            tuple(zeros16 for _ in range(8)))
        for o in range(8):
            sortidx_v[pl.ds(16 * o, 16)] = acc[o]

        pltpu.async_copy(c128_hbm.at[sortidx_v], crows_v, sem).wait()
        pltpu.sync_copy(crows_v, cents128_hbm.at[wid])
        pltpu.async_copy(pf_hbm.at[sortidx_v], rows_v, sem).wait()
        pltpu.sync_copy(rows_v, toks_hbm.at[wid])


def sc_select(key, pf, coords128):
    mesh = plsc.VectorSubcoreMesh(core_axis_name="c", subcore_axis_name="s")
    f = pl.kernel(
        _sc_body,
        mesh=mesh,
        out_type=[
            jax.ShapeDtypeStruct((B, MAX_TOKENS, TOKEN_DIM), _f32),
            jax.ShapeDtypeStruct((B, MAX_TOKENS, 128), _f32),
        ],
        scratch_types=[
            pltpu.VMEM((M + 16,), _i32),
            pltpu.VMEM((MAX_TOKENS + 32,), _i32),
            pltpu.VMEM((160,), _i32),
            pltpu.VMEM((MAX_TOKENS + 16,), _f32),
            pltpu.VMEM((MAX_TOKENS + 16,), _i32),
            pltpu.VMEM((MAX_TOKENS,), _i32),
            pltpu.VMEM((MAX_TOKENS, 128), _f32),
            pltpu.VMEM((MAX_TOKENS, TOKEN_DIM), _f32),
            pltpu.VMEM((96,), _i32),
            pltpu.SemaphoreType.DMA,
        ],
    )
    return f(key, pf, coords128)


def kernel(coordinates, features, ln_g, ln_b, Ws1, bs1, Ws2, bs2,
           Wm1, bm1, Wm2, bm2, Wm3, bm3, Wm4, bm4, Wi1, bi1, Wi2, bi2, tau):
    cf4 = coordinates[:, 1:5]
    pf, imp, key = _run_mlp(cf4, features, ln_g, ln_b, Ws1, bs1, Ws2, bs2,
                            Wm1, bm1, Wm2, bm2, Wm3, bm3, Wm4, bm4,
                            Wi1, bi1, Wi2, bi2)

    coords128 = jnp.concatenate([cf4, jnp.zeros((N, 124), _f32)], axis=1)
    toks, cents128 = sc_select(key, pf, coords128)
    cents = cents128[:, :, :4]
    masks = jnp.ones((B, MAX_TOKENS), dtype=bool)
    return toks, cents, masks


# speedup vs baseline: 1.0394x; 1.0394x over previous
import jax
import jax.numpy as jnp
from jax import lax
from jax.experimental import pallas as pl
from jax.experimental.pallas import tpu as pltpu, tpu_sc as plsc

N = 65536
B = 8
M = N // B
FEATURE_DIM = 256
MAX_TOKENS = 128
TOKEN_DIM = 768

ROWS = 1024

_bf = jnp.bfloat16
_f32 = jnp.float32
_i32 = jnp.int32
_u32 = jnp.uint32


def _bdot(x, w):
    return jnp.dot(x.astype(_bf), w, preferred_element_type=_f32)


def _mlp_body(cf_ref, feat_ref, lng_ref, lnb_ref,
              ws1_ref, bs1_ref, ws2_ref, bs2_ref,
              w1_ref, b1_ref,
              w2_ref, b2_ref, w3_ref, b3_ref, w4_ref, b4_ref,
              wi1_ref, bi1_ref, wi2_ref, bi2_ref,
              pf_ref, imp_ref, key_ref):
    x4 = cf_ref[...]
    mu = jnp.mean(x4, axis=1, keepdims=True)
    var = jnp.mean((x4 - mu) ** 2, axis=1, keepdims=True)
    cf = (x4 - mu) / jnp.sqrt(var + 1e-5) * lng_ref[...] + lnb_ref[...]

    sp = jnp.maximum(_bdot(cf, ws1_ref[...]) + bs1_ref[...], 0.0)
    sp = _bdot(sp, ws2_ref[...]) + bs2_ref[...]

    cat = jnp.concatenate([feat_ref[...].astype(_bf), sp.astype(_bf)], axis=1)
    h = jnp.maximum(jnp.dot(cat, w1_ref[...], preferred_element_type=_f32)
                    + b1_ref[...], 0.0)
    h = jnp.maximum(_bdot(h, w2_ref[...]) + b2_ref[...], 0.0)
    h = jnp.maximum(_bdot(h, w3_ref[...]) + b3_ref[...], 0.0)
    pf = _bdot(h, w4_ref[...]) + b4_ref[...]
    pf_ref[...] = pf

    t = jnp.maximum(_bdot(pf, wi1_ref[...]) + bi1_ref[...], 0.0)
    imp = _bdot(t, wi2_ref[...]) + bi2_ref[...]
    imp_ref[...] = imp

    bits = jax.lax.bitcast_convert_type(imp, jnp.int32)
    key = bits ^ (jnp.int32(-0x80000000) | (bits >> 31))
    key_ref[...] = key


def _run_mlp(cf4, features, ln_g, ln_b, Ws1, bs1, Ws2, bs2,
             Wm1, bm1, Wm2, bm2, Wm3, bm3, Wm4, bm4, Wi1, bi1, Wi2, bi2):
    grid = N // ROWS
    row_spec = lambda width: pl.BlockSpec((ROWS, width), lambda i: (i, 0))
    full = lambda a: pl.BlockSpec(a.shape, lambda i: (0,) * a.ndim)

    weights = [ln_g[None, :], ln_b[None, :],
               Ws1.T.astype(_bf), bs1[None, :],
               Ws2.T.astype(_bf), bs2[None, :],
               Wm1.T.astype(_bf), bm1[None, :],
               Wm2.T.astype(_bf), bm2[None, :],
               Wm3.T.astype(_bf), bm3[None, :],
               Wm4.T.astype(_bf), bm4[None, :],
               Wi1.T.astype(_bf), bi1[None, :],
               Wi2.T.astype(_bf), bi2[None, :]]

    pf, imp, key = pl.pallas_call(
        _mlp_body,
        grid=(grid,),
        in_specs=[row_spec(4), row_spec(FEATURE_DIM)] + [full(w) for w in weights],
        out_specs=[row_spec(TOKEN_DIM), row_spec(1), row_spec(1)],
        out_shape=[
            jax.ShapeDtypeStruct((N, TOKEN_DIM), jnp.float32),
            jax.ShapeDtypeStruct((N, 1), jnp.float32),
            jax.ShapeDtypeStruct((N, 1), jnp.int32),
        ],
    )(cf4, features, *weights)
    return pf, imp[:, 0], key[:, 0]


_VECS = M // 16


def _sc_body(key_hbm, pf_hbm, c128_hbm,
             toks_hbm, cents128_hbm,
             keys_v, selidx_v, eq_v, seltime_v, selkey_v,
             sortidx_v, crows_v, rows_v, sum_v,
             sem, sem2):
    wid = lax.axis_index("s") * 2 + lax.axis_index("c")

    @pl.when(wid < B)
    def _():
        lanes = lax.iota(_i32, 16)
        zeros16 = jnp.zeros((16,), _i32)
        ones16 = jnp.ones((16,), _i32)
        lane0 = lanes == 0

        def _sload(ref, p):
            return ref[pl.ds(p, 16)][0]

        def _sstore(ref, p, val, dtype):
            v = ref[pl.ds(p, 16)]
            ref[pl.ds(p, 16)] = jnp.where(lane0, jnp.full((16,), val, dtype),
                                          v)

        def _cstore(ref, p, val, flag, dtype):
            v = ref[pl.ds(p, 16)]
            new0 = jnp.where(flag > 0, val, v[0])
            ref[pl.ds(p, 16)] = jnp.where(lane0, jnp.full((16,), new0, dtype),
                                          v)

        def _vsum(vec):
            sum_v[pl.ds(0, 16)] = vec
            for off in (8, 4, 2, 1):
                sum_v[pl.ds(0, 16)] = (sum_v[pl.ds(0, 16)]
                                       + sum_v[pl.ds(off, 16)])
            return sum_v[pl.ds(0, 16)][0]

        def _popcnt(m):
            return _vsum(jnp.where(m, ones16, zeros16))

        pltpu.sync_copy(key_hbm.at[pl.ds(wid * M, M)],
                        keys_v.at[pl.ds(0, M)])

        def _count_ge(t):
            def body(i, cnt):
                kv = plsc.bitcast(keys_v[pl.ds(i * 16, 16)], _u32)
                return cnt + jnp.where(kv >= t, ones16, zeros16)
            return _vsum(lax.fori_loop(0, _VECS, body, zeros16))

        T = jnp.uint32(0)
        for bit in range(31, -1, -1):
            Ttry = T | jnp.uint32(1 << bit)
            T = jnp.where(_count_ge(Ttry) >= MAX_TOKENS, Ttry, T)
        need = MAX_TOKENS - _count_ge(T + jnp.uint32(1))

        def _compact(i, carry):
            kv = plsc.bitcast(keys_v[pl.ds(i * 16, 16)], _u32)
            m_gt = kv > T
            m_eq = kv == T
            any_hit = _popcnt(m_gt | m_eq) > 0

            def _hit(c):
                n_gt, n_eq = c
                sum_v[pl.ds(32, 16)] = jnp.where(m_gt, ones16, zeros16)
                sum_v[pl.ds(48, 16)] = jnp.where(m_eq, ones16, zeros16)

                def _lane(q, cc):
                    ng, ne = cc
                    fg = sum_v[pl.ds(32 + q, 16)][0]
                    fe = sum_v[pl.ds(48 + q, 16)][0]
                    iq = wid * M + i * 16 + q
                    _cstore(selidx_v, ng, iq, fg, _i32)
                    _cstore(eq_v, jnp.minimum(ne, 132), iq, fe, _i32)
                    return (ng + fg, ne + fe)
                return lax.fori_loop(0, 16, _lane, c)
            return lax.cond(any_hit, _hit, lambda c: c, carry)
        n_gt, _ = lax.fori_loop(0, _VECS, _compact, (jnp.int32(0),
                                                     jnp.int32(0)))

        def _fill_eq(j, _):
            _sstore(selidx_v, n_gt + j, _sload(eq_v, j), _i32)
            return 0
        lax.fori_loop(0, need, _fill_eq, 0)

        pltpu.async_copy(c128_hbm.at[selidx_v.at[pl.ds(0, MAX_TOKENS)]],
                         crows_v, sem).wait()

        def _lookup(p, _):
            tp = crows_v[p, pl.ds(0, 16)][3]
            _sstore(seltime_v, p, tp, _f32)
            _sstore(selkey_v, p,
                    _sload(keys_v, _sload(selidx_v, p) - wid * M), _i32)
            return 0
        lax.fori_loop(0, MAX_TOKENS, _lookup, 0)

        st = [seltime_v[pl.ds(16 * a, 16)] for a in range(8)]
        sk = [plsc.bitcast(selkey_v[pl.ds(16 * a, 16)], _u32)
              for a in range(8)]
        si = [selidx_v[pl.ds(16 * a, 16)] for a in range(8)]

        def _place(p, acc):
            tb = jnp.full((16,), _sload(seltime_v, p), _f32)
            kb = plsc.bitcast(jnp.full((16,), _sload(selkey_v, p), _i32),
                              _u32)
            ib = jnp.full((16,), _sload(selidx_v, p), _i32)
            cntv = zeros16
            for a in range(8):
                before = (st[a] < tb) | (
                    (st[a] == tb) & ((sk[a] > kb) | ((sk[a] == kb)
                                                     & (si[a] < ib))))
                cntv = cntv + jnp.where(before, ones16, zeros16)
            rank = _vsum(cntv)
            rb = jnp.full((16,), rank, _i32)
            return tuple(
                jnp.where(rb == (16 * o + lanes), ib, acc[o])
                for o in range(8))
        acc = lax.fori_loop(0, MAX_TOKENS, _place,
                            tuple(zeros16 for _ in range(8)))
        for o in range(8):
            sortidx_v[pl.ds(16 * o, 16)] = acc[o]

        cp1 = pltpu.async_copy(c128_hbm.at[sortidx_v], crows_v, sem)
        cp2 = pltpu.async_copy(pf_hbm.at[sortidx_v], rows_v, sem2)
        cp1.wait()
        pltpu.sync_copy(crows_v, cents128_hbm.at[wid])
        cp2.wait()
        pltpu.sync_copy(rows_v, toks_hbm.at[wid])


def sc_select(key, pf, coords128):
    mesh = plsc.VectorSubcoreMesh(core_axis_name="c", subcore_axis_name="s")
    f = pl.kernel(
        _sc_body,
        mesh=mesh,
        out_type=[
            jax.ShapeDtypeStruct((B, MAX_TOKENS, TOKEN_DIM), _f32),
            jax.ShapeDtypeStruct((B, MAX_TOKENS, 128), _f32),
        ],
        scratch_types=[
            pltpu.VMEM((M + 16,), _i32),
            pltpu.VMEM((MAX_TOKENS + 32,), _i32),
            pltpu.VMEM((160,), _i32),
            pltpu.VMEM((MAX_TOKENS + 16,), _f32),
            pltpu.VMEM((MAX_TOKENS + 16,), _i32),
            pltpu.VMEM((MAX_TOKENS,), _i32),
            pltpu.VMEM((MAX_TOKENS, 128), _f32),
            pltpu.VMEM((MAX_TOKENS, TOKEN_DIM), _f32),
            pltpu.VMEM((96,), _i32),
            pltpu.SemaphoreType.DMA,
            pltpu.SemaphoreType.DMA,
        ],
    )
    return f(key, pf, coords128)


def kernel(coordinates, features, ln_g, ln_b, Ws1, bs1, Ws2, bs2,
           Wm1, bm1, Wm2, bm2, Wm3, bm3, Wm4, bm4, Wi1, bi1, Wi2, bi2, tau):
    cf4 = coordinates[:, 1:5]
    pf, imp, key = _run_mlp(cf4, features, ln_g, ln_b, Ws1, bs1, Ws2, bs2,
                            Wm1, bm1, Wm2, bm2, Wm3, bm3, Wm4, bm4,
                            Wi1, bi1, Wi2, bi2)

    coords128 = jnp.concatenate([cf4, jnp.zeros((N, 124), _f32)], axis=1)
    toks, cents128 = sc_select(key, pf, coords128)
    cents = cents128[:, :, :4]
    masks = jnp.ones((B, MAX_TOKENS), dtype=bool)
    return toks, cents, masks


# speedup vs baseline: 1.1911x; 1.1460x over previous
import jax
import jax.numpy as jnp
from jax import lax
from jax.experimental import pallas as pl
from jax.experimental.pallas import tpu as pltpu, tpu_sc as plsc

N = 65536
B = 8
M = N // B
FEATURE_DIM = 256
MAX_TOKENS = 128
TOKEN_DIM = 768

ROWS = 2048

_bf = jnp.bfloat16
_f32 = jnp.float32
_i32 = jnp.int32
_u32 = jnp.uint32


def _bdot(x, w):
    return jnp.dot(x.astype(_bf), w, preferred_element_type=_f32)


def _mlp_body(cf_ref, feat_ref, lng_ref, lnb_ref,
              ws1_ref, bs1_ref, ws2_ref, bs2_ref,
              w1_ref, b1_ref,
              w2_ref, b2_ref, w3_ref, b3_ref, w4_ref, b4_ref,
              wi1_ref, bi1_ref, wi2_ref, bi2_ref,
              pf_ref, key_ref):
    x4 = cf_ref[...]
    mu = jnp.mean(x4, axis=1, keepdims=True)
    var = jnp.mean((x4 - mu) ** 2, axis=1, keepdims=True)
    cf = (x4 - mu) / jnp.sqrt(var + 1e-5) * lng_ref[...] + lnb_ref[...]

    sp = jnp.maximum(_bdot(cf, ws1_ref[...]) + bs1_ref[...], 0.0)
    sp = _bdot(sp, ws2_ref[...]) + bs2_ref[...]

    cat = jnp.concatenate([feat_ref[...].astype(_bf), sp.astype(_bf)], axis=1)
    h = jnp.maximum(jnp.dot(cat, w1_ref[...], preferred_element_type=_f32)
                    + b1_ref[...], 0.0)
    h = jnp.maximum(_bdot(h, w2_ref[...]) + b2_ref[...], 0.0)
    h = jnp.maximum(_bdot(h, w3_ref[...]) + b3_ref[...], 0.0)
    pf = _bdot(h, w4_ref[...]) + b4_ref[...]
    pf_ref[...] = pf

    t = jnp.maximum(_bdot(pf, wi1_ref[...]) + bi1_ref[...], 0.0)
    imp = _bdot(t, wi2_ref[...]) + bi2_ref[...]

    bits = jax.lax.bitcast_convert_type(imp, jnp.int32)
    key = bits ^ (jnp.int32(-0x80000000) | (bits >> 31))
    key_ref[...] = key


def _run_mlp(cf4, features, ln_g, ln_b, Ws1, bs1, Ws2, bs2,
             Wm1, bm1, Wm2, bm2, Wm3, bm3, Wm4, bm4, Wi1, bi1, Wi2, bi2):
    grid = N // ROWS
    row_spec = lambda width: pl.BlockSpec((ROWS, width), lambda i: (i, 0))
    full = lambda a: pl.BlockSpec(a.shape, lambda i: (0,) * a.ndim)

    weights = [ln_g[None, :], ln_b[None, :],
               Ws1.T.astype(_bf), bs1[None, :],
               Ws2.T.astype(_bf), bs2[None, :],
               Wm1.T.astype(_bf), bm1[None, :],
               Wm2.T.astype(_bf), bm2[None, :],
               Wm3.T.astype(_bf), bm3[None, :],
               Wm4.T.astype(_bf), bm4[None, :],
               Wi1.T.astype(_bf), bi1[None, :],
               Wi2.T.astype(_bf), bi2[None, :]]

    pf, key = pl.pallas_call(
        _mlp_body,
        grid=(grid,),
        in_specs=[row_spec(4), row_spec(FEATURE_DIM)] + [full(w) for w in weights],
        out_specs=[row_spec(TOKEN_DIM), row_spec(1)],
        out_shape=[
            jax.ShapeDtypeStruct((N, TOKEN_DIM), jnp.float32),
            jax.ShapeDtypeStruct((N, 1), jnp.int32),
        ],
    )(cf4, features, *weights)
    return pf, key[:, 0]


_VECS = M // 16


def _sc_body(key_hbm, pf_hbm, c128_hbm,
             toks_hbm, cents128_hbm,
             keys_v, selidx_v, eq_v, seltime_v, selkey_v,
             sortidx_v, crows_v, rows_v, sum_v,
             sem, sem2):
    wid = lax.axis_index("s") * 2 + lax.axis_index("c")

    @pl.when(wid < B)
    def _():
        lanes = lax.iota(_i32, 16)
        zeros16 = jnp.zeros((16,), _i32)
        ones16 = jnp.ones((16,), _i32)
        lane0 = lanes == 0

        def _sload(ref, p):
            return ref[pl.ds(p, 16)][0]

        def _sstore(ref, p, val, dtype):
            v = ref[pl.ds(p, 16)]
            ref[pl.ds(p, 16)] = jnp.where(lane0, jnp.full((16,), val, dtype),
                                          v)

        def _cstore(ref, p, val, flag, dtype):
            v = ref[pl.ds(p, 16)]
            new0 = jnp.where(flag > 0, val, v[0])
            ref[pl.ds(p, 16)] = jnp.where(lane0, jnp.full((16,), new0, dtype),
                                          v)

        def _vsum(vec):
            sum_v[pl.ds(0, 16)] = vec
            for off in (8, 4, 2, 1):
                sum_v[pl.ds(0, 16)] = (sum_v[pl.ds(0, 16)]
                                       + sum_v[pl.ds(off, 16)])
            return sum_v[pl.ds(0, 16)][0]

        def _popcnt(m):
            return _vsum(jnp.where(m, ones16, zeros16))

        pltpu.sync_copy(key_hbm.at[pl.ds(wid * M, M)],
                        keys_v.at[pl.ds(0, M)])

        def _count_ge(t):
            def body(i, cnt):
                for u in range(4):
                    kv = plsc.bitcast(keys_v[pl.ds(i * 64 + u * 16, 16)],
                                      _u32)
                    cnt = cnt + jnp.where(kv >= t, ones16, zeros16)
                return cnt
            return _vsum(lax.fori_loop(0, _VECS // 4, body, zeros16))

        def _minmax(i, mm):
            mn, mx = mm
            for u in range(4):
                kv = plsc.bitcast(keys_v[pl.ds(i * 64 + u * 16, 16)], _u32)
                mn = jnp.minimum(mn, kv)
                mx = jnp.maximum(mx, kv)
            return mn, mx
        mnv, mxv = lax.fori_loop(0, _VECS // 4, _minmax,
                                 (jnp.full((16,), 0xFFFFFFFF, _u32),
                                  jnp.zeros((16,), _u32)))
        sum_v[pl.ds(0, 16)] = plsc.bitcast(mnv, _i32)
        for off in (8, 4, 2, 1):
            a = plsc.bitcast(sum_v[pl.ds(0, 16)], _u32)
            b = plsc.bitcast(sum_v[pl.ds(off, 16)], _u32)
            sum_v[pl.ds(0, 16)] = plsc.bitcast(jnp.minimum(a, b), _i32)
        mink = plsc.bitcast(sum_v[pl.ds(0, 16)], _u32)[0]
        sum_v[pl.ds(0, 16)] = plsc.bitcast(mxv, _i32)
        for off in (8, 4, 2, 1):
            a = plsc.bitcast(sum_v[pl.ds(0, 16)], _u32)
            b = plsc.bitcast(sum_v[pl.ds(off, 16)], _u32)
            sum_v[pl.ds(0, 16)] = plsc.bitcast(jnp.maximum(a, b), _i32)
        maxk = plsc.bitcast(sum_v[pl.ds(0, 16)], _u32)[0]

        T = jnp.uint32(0)
        for bit in range(31, -1, -1):
            Ttry = T | jnp.uint32(1 << bit)
            cnt = lax.cond(
                Ttry > maxk, lambda t: jnp.int32(0),
                lambda t: lax.cond(t <= mink, lambda t2: jnp.int32(M),
                                   _count_ge, t), Ttry)
            T = jnp.where(cnt >= MAX_TOKENS, Ttry, T)
        need = MAX_TOKENS - lax.cond(T >= maxk, lambda t: jnp.int32(0),
                                     _count_ge, T + jnp.uint32(1))

        def _compact(i, carry):
            kv = plsc.bitcast(keys_v[pl.ds(i * 16, 16)], _u32)
            m_gt = kv > T
            m_eq = kv == T
            any_hit = _popcnt(m_gt | m_eq) > 0

            def _hit(c):
                n_gt, n_eq = c
                sum_v[pl.ds(32, 16)] = jnp.where(m_gt, ones16, zeros16)
                sum_v[pl.ds(48, 16)] = jnp.where(m_eq, ones16, zeros16)

                def _lane(q, cc):
                    ng, ne = cc
                    fg = sum_v[pl.ds(32 + q, 16)][0]
                    fe = sum_v[pl.ds(48 + q, 16)][0]
                    iq = wid * M + i * 16 + q
                    _cstore(selidx_v, ng, iq, fg, _i32)
                    _cstore(eq_v, jnp.minimum(ne, 132), iq, fe, _i32)
                    return (ng + fg, ne + fe)
                return lax.fori_loop(0, 16, _lane, c)
            return lax.cond(any_hit, _hit, lambda c: c, carry)
        n_gt, _ = lax.fori_loop(0, _VECS, _compact, (jnp.int32(0),
                                                     jnp.int32(0)))

        def _fill_eq(j, _):
            _sstore(selidx_v, n_gt + j, _sload(eq_v, j), _i32)
            return 0
        lax.fori_loop(0, need, _fill_eq, 0)

        pltpu.async_copy(c128_hbm.at[selidx_v.at[pl.ds(0, MAX_TOKENS)]],
                         crows_v, sem).wait()

        def _lookup(p, _):
            tp = crows_v[p, pl.ds(0, 16)][3]
            _sstore(seltime_v, p, tp, _f32)
            _sstore(selkey_v, p,
                    _sload(keys_v, _sload(selidx_v, p) - wid * M), _i32)
            return 0
        lax.fori_loop(0, MAX_TOKENS, _lookup, 0)

        st = [seltime_v[pl.ds(16 * a, 16)] for a in range(8)]
        sk = [plsc.bitcast(selkey_v[pl.ds(16 * a, 16)], _u32)
              for a in range(8)]
        si = [selidx_v[pl.ds(16 * a, 16)] for a in range(8)]

        def _place(p, acc):
            tb = jnp.full((16,), _sload(seltime_v, p), _f32)
            kb = plsc.bitcast(jnp.full((16,), _sload(selkey_v, p), _i32),
                              _u32)
            ib = jnp.full((16,), _sload(selidx_v, p), _i32)
            cntv = zeros16
            for a in range(8):
                before = (st[a] < tb) | (
                    (st[a] == tb) & ((sk[a] > kb) | ((sk[a] == kb)
                                                     & (si[a] < ib))))
                cntv = cntv + jnp.where(before, ones16, zeros16)
            rank = _vsum(cntv)
            rb = jnp.full((16,), rank, _i32)
            return tuple(
                jnp.where(rb == (16 * o + lanes), ib, acc[o])
                for o in range(8))
        acc = lax.fori_loop(0, MAX_TOKENS, _place,
                            tuple(zeros16 for _ in range(8)))
        for o in range(8):
            sortidx_v[pl.ds(16 * o, 16)] = acc[o]

        cp1 = pltpu.async_copy(c128_hbm.at[sortidx_v], crows_v, sem)
        cp2 = pltpu.async_copy(pf_hbm.at[sortidx_v], rows_v, sem2)
        cp1.wait()
        pltpu.sync_copy(crows_v, cents128_hbm.at[wid])
        cp2.wait()
        pltpu.sync_copy(rows_v, toks_hbm.at[wid])


def sc_select(key, pf, coords128):
    mesh = plsc.VectorSubcoreMesh(core_axis_name="c", subcore_axis_name="s")
    f = pl.kernel(
        _sc_body,
        mesh=mesh,
        out_type=[
            jax.ShapeDtypeStruct((B, MAX_TOKENS, TOKEN_DIM), _f32),
            jax.ShapeDtypeStruct((B, MAX_TOKENS, 128), _f32),
        ],
        scratch_types=[
            pltpu.VMEM((M + 16,), _i32),
            pltpu.VMEM((MAX_TOKENS + 32,), _i32),
            pltpu.VMEM((160,), _i32),
            pltpu.VMEM((MAX_TOKENS + 16,), _f32),
            pltpu.VMEM((MAX_TOKENS + 16,), _i32),
            pltpu.VMEM((MAX_TOKENS,), _i32),
            pltpu.VMEM((MAX_TOKENS, 128), _f32),
            pltpu.VMEM((MAX_TOKENS, TOKEN_DIM), _f32),
            pltpu.VMEM((96,), _i32),
            pltpu.SemaphoreType.DMA,
            pltpu.SemaphoreType.DMA,
        ],
    )
    return f(key, pf, coords128)


def kernel(coordinates, features, ln_g, ln_b, Ws1, bs1, Ws2, bs2,
           Wm1, bm1, Wm2, bm2, Wm3, bm3, Wm4, bm4, Wi1, bi1, Wi2, bi2, tau):
    cf4 = coordinates[:, 1:5]
    pf, key = _run_mlp(cf4, features, ln_g, ln_b, Ws1, bs1, Ws2, bs2,
                       Wm1, bm1, Wm2, bm2, Wm3, bm3, Wm4, bm4,
                       Wi1, bi1, Wi2, bi2)

    coords128 = jnp.concatenate([cf4, jnp.zeros((N, 124), _f32)], axis=1)
    toks, cents128 = sc_select(key, pf, coords128)
    cents = cents128[:, :, :4]
    masks = jnp.ones((B, MAX_TOKENS), dtype=bool)
    return toks, cents, masks
